# baseline (speedup 1.0000x reference)
import functools

import jax
import jax.numpy as jnp
from jax import lax
from jax.experimental import pallas as pl
from jax.experimental.pallas import tpu as pltpu
from jax.experimental.pallas import tpu_sc as plsc

POOLED = 7
GROUP = 7
SCALE = 0.0625
SAMPLE = 4
B, C, H, W = 2, 490, 64, 64
OC = C // (GROUP * GROUP)
NROIS = 1024
NG = POOLED * POOLED
PLANE = H * W
PLANES_WORDS = B * OC * PLANE
NC, NS, L = 2, 16, 16
NW = NC * NS
NB = NROIS // L
ITEMS = NG * NB
ITEMS_PER_W = ITEMS // NW
RING = 4


def _pack_body(x_ref, out_ref):
    x = x_ref[...]
    xs = jnp.concatenate([x[..., 1:], x[..., -1:]], axis=-1)
    lo = jax.lax.bitcast_convert_type(x.astype(jnp.bfloat16),
                                      jnp.uint16).astype(jnp.uint32)
    hi = jax.lax.bitcast_convert_type(xs.astype(jnp.bfloat16),
                                      jnp.uint16).astype(jnp.uint32)
    word = jax.lax.bitcast_convert_type(lo | (hi << 16), jnp.int32)
    w4 = word.reshape(NG * H * W // 128, 2, 64)
    out_ref[...] = jnp.concatenate([w4[:, 0, :], w4[:, 1, :]], axis=-1)


def _params_body(rois_ref, out_ref):
    r = rois_ref[...]
    b = r[0]
    cx = r[1] * SCALE
    cy = r[2] * SCALE
    rw = jnp.maximum(r[3] * SCALE, 0.1)
    rh = jnp.maximum(r[4] * SCALE, 0.1)
    theta = r[5]
    bin_w = rw / POOLED
    bin_h = rh / POOLED
    half_w = rw / 2.0
    half_h = rh / 2.0
    cos_t = jnp.cos(theta)
    sin_t = jnp.sin(theta)
    rows = jnp.stack([b, cx, cy, bin_w, bin_h, half_w, half_h, cos_t, sin_t])
    pad = jnp.zeros((16 - rows.shape[0], NROIS), jnp.float32)
    out_ref[...] = jnp.concatenate([rows, pad], axis=0)


def _sc_body(feat_hbm, params_hbm, out_hbm, planes_v, params_v,
             ov0, ov1, ov2, ov3, sem_stage, sem_out):
    outs = [ov0, ov1, ov2, ov3]
    wid = lax.axis_index("s") * NC + lax.axis_index("c")
    pltpu.sync_copy(params_hbm, params_v)
    item0 = wid * ITEMS_PER_W
    g0 = item0 // NB
    g1 = (item0 + ITEMS_PER_W - 1) // NB

    def g_body(g, _):
        with jax.named_scope("stage_planes"):
            handles = []
            for bb in range(B):
                for oc in range(OC):
                    src = (bb * C + oc * GROUP * GROUP) * PLANE
                    h = pltpu.async_copy(
                        feat_hbm.at[pl.ds(g * PLANE + src, PLANE)],
                        planes_v.at[pl.ds((oc * B + bb) * PLANE, PLANE)],
                        sem_stage)
                    handles.append(h)
            for h in handles:
                h.wait()

        rb0 = jnp.maximum(item0 - g * NB, 0)
        rb1 = jnp.minimum(item0 + ITEMS_PER_W - g * NB, NB)
        ph_f = (g // POOLED).astype(jnp.float32)
        pw_f = (g % POOLED).astype(jnp.float32)

        @plsc.parallel_loop(rb0, rb1, 1, unroll=1)
        def rb_body(rb):
            item = g * NB + rb
            slot = item % RING

            for k in range(RING):
                @pl.when((slot == k) & (item - item0 >= RING))
                def _(k=k):
                    pltpu.make_async_copy(out_hbm.at[g, rb],
                                          outs[k], sem_out.at[k]).wait()

            off = rb * L
            b_f = params_v[0, pl.ds(off, L)]
            cx = params_v[1, pl.ds(off, L)]
            cy = params_v[2, pl.ds(off, L)]
            bin_w = params_v[3, pl.ds(off, L)]
            bin_h = params_v[4, pl.ds(off, L)]
            half_w = params_v[5, pl.ds(off, L)]
            half_h = params_v[6, pl.ds(off, L)]
            cos_t = params_v[7, pl.ds(off, L)]
            sin_t = params_v[8, pl.ds(off, L)]
            base = b_f.astype(jnp.int32) * PLANE
            acc = [jnp.zeros((L,), jnp.float32) for _ in range(OC)]
            for s in range(SAMPLE * SAMPLE):
                sy_c = (s // SAMPLE + 0.5) / SAMPLE
                sx_c = (s % SAMPLE + 0.5) / SAMPLE
                xx = (pw_f + sx_c) * bin_w - half_w
                yy = (ph_f + sy_c) * bin_h - half_h
                x = xx * cos_t - yy * sin_t + cx
                y = xx * sin_t + yy * cos_t + cy
                valid = (y > -1.0) & (y < float(H)) & (x > -1.0) & (x < float(W))
                yc = jnp.clip(y, 0.0, H - 1.0)
                xc = jnp.clip(x, 0.0, W - 1.0)
                y0 = yc.astype(jnp.int32)
                x0 = xc.astype(jnp.int32)
                y1 = jnp.minimum(y0 + 1, H - 1)
                ly = yc - y0.astype(jnp.float32)
                lx = xc - x0.astype(jnp.float32)
                hy = 1.0 - ly
                hx = 1.0 - lx
                scl = jnp.where(valid, 1.0 / (SAMPLE * SAMPLE), 0.0)
                hy = hy * scl
                ly = ly * scl
                w00 = hy * hx
                w01 = hy * lx
                w10 = ly * hx
                w11 = ly * lx
                i00 = base + y0 * W + x0
                i10 = base + y1 * W + x0
                wtop = plsc.pack(w00, w01, format=plsc.PackFormat.INTERLEAVED)
                wbot = plsc.pack(w10, w11, format=plsc.PackFormat.INTERLEAVED)
                mask_hi = jnp.full((L,), -65536, jnp.int32)
                for oc in range(OC):
                    ref = planes_v.at[pl.ds(oc * B * PLANE, B * PLANE)]
                    gt = plsc.bitcast(plsc.load_gather(ref, [i00]),
                                      jnp.bfloat16)
                    gb = plsc.bitcast(plsc.load_gather(ref, [i10]),
                                      jnp.bfloat16)
                    sm = plsc.bitcast(gt * wtop + gb * wbot, jnp.int32)
                    lo = plsc.bitcast(sm << 16, jnp.float32)
                    hi = plsc.bitcast(sm & mask_hi, jnp.float32)
                    acc[oc] = acc[oc] + (lo + hi)
            for k in range(RING):
                @pl.when(slot == k)
                def _(k=k):
                    for oc in range(OC):
                        outs[k][oc, :] = acc[oc]
                    pltpu.async_copy(outs[k], out_hbm.at[g, rb],
                                     sem_out.at[k])

        return 0

    lax.fori_loop(g0, g1 + 1, g_body, 0)

    for k in range(RING):
        pltpu.make_async_copy(out_hbm.at[0, k], outs[k], sem_out.at[k]).wait()


def kernel(features, rois):
    rois_t = rois.T

    nrows = B * C * H * W // 128
    chunk = NG * H * W // 128
    packed = pl.pallas_call(
        _pack_body,
        grid=(B, OC),
        in_specs=[pl.BlockSpec((1, NG, H, W), lambda b, k: (b, k, 0, 0))],
        out_specs=pl.BlockSpec((chunk, 128), lambda b, k: (b * OC + k, 0)),
        out_shape=jax.ShapeDtypeStruct((nrows, 128), jnp.int32),
    )(features)
    feat_flat = packed.reshape(B * C * PLANE)

    params = pl.pallas_call(
        _params_body,
        out_shape=jax.ShapeDtypeStruct((16, NROIS), jnp.float32),
    )(rois_t)

    mesh = plsc.VectorSubcoreMesh(core_axis_name="c", subcore_axis_name="s",
                                  num_cores=NC, num_subcores=NS)
    sc = functools.partial(
        pl.kernel,
        mesh=mesh,
        out_type=jax.ShapeDtypeStruct((NG, NB, OC, L), jnp.float32),
        scratch_types=[
            pltpu.VMEM((PLANES_WORDS,), jnp.int32),
            pltpu.VMEM((16, NROIS), jnp.float32),
            pltpu.VMEM((OC, L), jnp.float32),
            pltpu.VMEM((OC, L), jnp.float32),
            pltpu.VMEM((OC, L), jnp.float32),
            pltpu.VMEM((OC, L), jnp.float32),
            pltpu.SemaphoreType.DMA,
            pltpu.SemaphoreType.DMA((RING,)),
        ],
        compiler_params=pltpu.CompilerParams(needs_layout_passes=False),
    )(_sc_body)
    out = sc(feat_flat, params)

    out = out.transpose(1, 3, 2, 0).reshape(NROIS, OC, POOLED, POOLED)
    return out


# speedup vs baseline: 1.0753x; 1.0753x over previous
import functools

import jax
import jax.numpy as jnp
from jax import lax
from jax.experimental import pallas as pl
from jax.experimental.pallas import tpu as pltpu
from jax.experimental.pallas import tpu_sc as plsc

POOLED = 7
GROUP = 7
SCALE = 0.0625
SAMPLE = 4
B, C, H, W = 2, 490, 64, 64
OC = C // (GROUP * GROUP)
NROIS = 1024
NG = POOLED * POOLED
PLANE = H * W
PLANES_WORDS = B * OC * PLANE
NC, NS, L = 2, 16, 16
NW = NC * NS
NB = NROIS // L
ITEMS = NG * NB
ITEMS_PER_W = ITEMS // NW
RING = 4


def _pack_body(x_ref, out_ref):
    x = x_ref[...][0]
    xt = jnp.transpose(x, (2, 0, 1))
    xs = jnp.concatenate([xt[..., 1:], xt[..., -1:]], axis=-1)
    lo = jax.lax.bitcast_convert_type(xt.astype(jnp.bfloat16),
                                      jnp.uint16).astype(jnp.uint32)
    hi = jax.lax.bitcast_convert_type(xs.astype(jnp.bfloat16),
                                      jnp.uint16).astype(jnp.uint32)
    word = jax.lax.bitcast_convert_type(lo | (hi << 16), jnp.int32)
    w4 = word.reshape(C, 8, 2, 64)
    out_ref[...] = jnp.concatenate(
        [w4[:, :, 0, :], w4[:, :, 1, :]], axis=-1)[None]


def _params_body(rois_ref, out_ref):
    r = rois_ref[...]
    b = r[0]
    cx = r[1] * SCALE
    cy = r[2] * SCALE
    rw = jnp.maximum(r[3] * SCALE, 0.1)
    rh = jnp.maximum(r[4] * SCALE, 0.1)
    theta = r[5]
    bin_w = rw / POOLED
    bin_h = rh / POOLED
    half_w = rw / 2.0
    half_h = rh / 2.0
    cos_t = jnp.cos(theta)
    sin_t = jnp.sin(theta)
    rows = jnp.stack([b, cx, cy, bin_w, bin_h, half_w, half_h, cos_t, sin_t])
    pad = jnp.zeros((16 - rows.shape[0], NROIS), jnp.float32)
    out_ref[...] = jnp.concatenate([rows, pad], axis=0)


def _sc_body(feat_hbm, params_hbm, out_hbm, planes_v, params_v,
             ov0, ov1, ov2, ov3, sem_stage, sem_out):
    outs = [ov0, ov1, ov2, ov3]
    wid = lax.axis_index("s") * NC + lax.axis_index("c")
    pltpu.sync_copy(params_hbm, params_v)
    item0 = wid * ITEMS_PER_W
    g0 = item0 // NB
    g1 = (item0 + ITEMS_PER_W - 1) // NB

    def g_body(g, _):
        with jax.named_scope("stage_planes"):
            handles = []
            for bb in range(B):
                for oc in range(OC):
                    src = (bb * C + oc * GROUP * GROUP) * PLANE
                    h = pltpu.async_copy(
                        feat_hbm.at[pl.ds(g * PLANE + src, PLANE)],
                        planes_v.at[pl.ds((oc * B + bb) * PLANE, PLANE)],
                        sem_stage)
                    handles.append(h)
            for h in handles:
                h.wait()

        rb0 = jnp.maximum(item0 - g * NB, 0)
        rb1 = jnp.minimum(item0 + ITEMS_PER_W - g * NB, NB)
        ph_f = (g // POOLED).astype(jnp.float32)
        pw_f = (g % POOLED).astype(jnp.float32)

        @plsc.parallel_loop(rb0, rb1, 1, unroll=1)
        def rb_body(rb):
            item = g * NB + rb
            slot = item % RING

            for k in range(RING):
                @pl.when((slot == k) & (item - item0 >= RING))
                def _(k=k):
                    pltpu.make_async_copy(out_hbm.at[g, rb],
                                          outs[k], sem_out.at[k]).wait()

            off = rb * L
            b_f = params_v[0, pl.ds(off, L)]
            cx = params_v[1, pl.ds(off, L)]
            cy = params_v[2, pl.ds(off, L)]
            bin_w = params_v[3, pl.ds(off, L)]
            bin_h = params_v[4, pl.ds(off, L)]
            half_w = params_v[5, pl.ds(off, L)]
            half_h = params_v[6, pl.ds(off, L)]
            cos_t = params_v[7, pl.ds(off, L)]
            sin_t = params_v[8, pl.ds(off, L)]
            base = b_f.astype(jnp.int32) * PLANE
            acc = [jnp.zeros((L,), jnp.float32) for _ in range(OC)]
            for s in range(SAMPLE * SAMPLE):
                sy_c = (s // SAMPLE + 0.5) / SAMPLE
                sx_c = (s % SAMPLE + 0.5) / SAMPLE
                xx = (pw_f + sx_c) * bin_w - half_w
                yy = (ph_f + sy_c) * bin_h - half_h
                x = xx * cos_t - yy * sin_t + cx
                y = xx * sin_t + yy * cos_t + cy
                valid = (y > -1.0) & (y < float(H)) & (x > -1.0) & (x < float(W))
                yc = jnp.clip(y, 0.0, H - 1.0)
                xc = jnp.clip(x, 0.0, W - 1.0)
                y0 = yc.astype(jnp.int32)
                x0 = xc.astype(jnp.int32)
                y1 = jnp.minimum(y0 + 1, H - 1)
                ly = yc - y0.astype(jnp.float32)
                lx = xc - x0.astype(jnp.float32)
                hy = 1.0 - ly
                hx = 1.0 - lx
                scl = jnp.where(valid, 1.0 / (SAMPLE * SAMPLE), 0.0)
                hy = hy * scl
                ly = ly * scl
                w00 = hy * hx
                w01 = hy * lx
                w10 = ly * hx
                w11 = ly * lx
                i00 = base + y0 * W + x0
                i10 = base + y1 * W + x0
                wtop = plsc.pack(w00, w01, format=plsc.PackFormat.INTERLEAVED)
                wbot = plsc.pack(w10, w11, format=plsc.PackFormat.INTERLEAVED)
                mask_hi = jnp.full((L,), -65536, jnp.int32)
                for oc in range(OC):
                    ref = planes_v.at[pl.ds(oc * B * PLANE, B * PLANE)]
                    gt = plsc.bitcast(plsc.load_gather(ref, [i00]),
                                      jnp.bfloat16)
                    gb = plsc.bitcast(plsc.load_gather(ref, [i10]),
                                      jnp.bfloat16)
                    sm = plsc.bitcast(gt * wtop + gb * wbot, jnp.int32)
                    lo = plsc.bitcast(sm << 16, jnp.float32)
                    hi = plsc.bitcast(sm & mask_hi, jnp.float32)
                    acc[oc] = acc[oc] + (lo + hi)
            for k in range(RING):
                @pl.when(slot == k)
                def _(k=k):
                    for oc in range(OC):
                        outs[k][oc, :] = acc[oc]
                    pltpu.async_copy(outs[k], out_hbm.at[g, rb],
                                     sem_out.at[k])

        return 0

    lax.fori_loop(g0, g1 + 1, g_body, 0)

    for k in range(RING):
        pltpu.make_async_copy(out_hbm.at[0, k], outs[k], sem_out.at[k]).wait()


def kernel(features, rois):
    rois_t = rois.T

    feat_t = features.transpose(0, 2, 3, 1)
    yblk = 16
    packed = pl.pallas_call(
        _pack_body,
        grid=(B, H // yblk),
        in_specs=[pl.BlockSpec((1, yblk, W, C), lambda b, i: (b, i, 0, 0))],
        out_specs=pl.BlockSpec((1, C, yblk // 2, 128),
                               lambda b, i: (b, 0, i, 0)),
        out_shape=jax.ShapeDtypeStruct((B, C, H * W // 128, 128), jnp.int32),
    )(feat_t)
    feat_flat = packed.reshape(B * C * PLANE)

    params = pl.pallas_call(
        _params_body,
        out_shape=jax.ShapeDtypeStruct((16, NROIS), jnp.float32),
    )(rois_t)

    mesh = plsc.VectorSubcoreMesh(core_axis_name="c", subcore_axis_name="s",
                                  num_cores=NC, num_subcores=NS)
    sc = functools.partial(
        pl.kernel,
        mesh=mesh,
        out_type=jax.ShapeDtypeStruct((NG, NB, OC, L), jnp.float32),
        scratch_types=[
            pltpu.VMEM((PLANES_WORDS,), jnp.int32),
            pltpu.VMEM((16, NROIS), jnp.float32),
            pltpu.VMEM((OC, L), jnp.float32),
            pltpu.VMEM((OC, L), jnp.float32),
            pltpu.VMEM((OC, L), jnp.float32),
            pltpu.VMEM((OC, L), jnp.float32),
            pltpu.SemaphoreType.DMA,
            pltpu.SemaphoreType.DMA((RING,)),
        ],
        compiler_params=pltpu.CompilerParams(needs_layout_passes=False),
    )(_sc_body)
    out = sc(feat_flat, params)

    out = out.transpose(1, 3, 2, 0).reshape(NROIS, OC, POOLED, POOLED)
    return out


# speedup vs baseline: 1.2302x; 1.1440x over previous
import functools

import jax
import jax.numpy as jnp
from jax import lax
from jax.experimental import pallas as pl
from jax.experimental.pallas import tpu as pltpu
from jax.experimental.pallas import tpu_sc as plsc

POOLED = 7
GROUP = 7
SCALE = 0.0625
SAMPLE = 4
B, C, H, W = 2, 490, 64, 64
OC = C // (GROUP * GROUP)
NROIS = 1024
NG = POOLED * POOLED
PLANE = H * W
PLANES_WORDS = B * OC * PLANE
NC, NS, L = 2, 16, 16
NW = NC * NS
NB = NROIS // L
ITEMS = NG * NB
ITEMS_PER_W = ITEMS // NW
RING = 4


def _pack_body(x_ref, out_ref):
    x = x_ref[...][0]
    xt = jnp.transpose(x, (2, 0, 1))
    xs = jnp.concatenate([xt[..., 1:], xt[..., -1:]], axis=-1)
    lo = jax.lax.bitcast_convert_type(xt.astype(jnp.bfloat16),
                                      jnp.uint16).astype(jnp.uint32)
    hi = jax.lax.bitcast_convert_type(xs.astype(jnp.bfloat16),
                                      jnp.uint16).astype(jnp.uint32)
    word = jax.lax.bitcast_convert_type(lo | (hi << 16), jnp.int32)
    w4 = word.reshape(C, 8, 2, 64)
    out_ref[...] = jnp.concatenate(
        [w4[:, :, 0, :], w4[:, :, 1, :]], axis=-1)[None]


def _params_body(rois_ref, out_ref):
    r = rois_ref[...]
    b = r[0]
    cx = r[1] * SCALE
    cy = r[2] * SCALE
    rw = jnp.maximum(r[3] * SCALE, 0.1)
    rh = jnp.maximum(r[4] * SCALE, 0.1)
    theta = r[5]
    bin_w = rw / POOLED
    bin_h = rh / POOLED
    half_w = rw / 2.0
    half_h = rh / 2.0
    cos_t = jnp.cos(theta)
    sin_t = jnp.sin(theta)
    rows = jnp.stack([b, cx, cy, bin_w, bin_h, half_w, half_h, cos_t, sin_t])
    pad = jnp.zeros((16 - rows.shape[0], NROIS), jnp.float32)
    out_ref[...] = jnp.concatenate([rows, pad], axis=0)


def _sc_body(feat_hbm, params_hbm, out_hbm, planes_v, params_v,
             ov0, ov1, ov2, ov3, sem_stage, sem_out):
    outs = [ov0, ov1, ov2, ov3]
    wid = lax.axis_index("s") * NC + lax.axis_index("c")
    pltpu.sync_copy(params_hbm, params_v)
    item0 = wid * ITEMS_PER_W
    g0 = item0 // NB
    g1 = (item0 + ITEMS_PER_W - 1) // NB

    def g_body(g, _):
        with jax.named_scope("stage_planes"):
            handles = []
            for bb in range(B):
                for oc in range(OC):
                    src = (bb * C + oc * GROUP * GROUP) * PLANE
                    h = pltpu.async_copy(
                        feat_hbm.at[pl.ds(g * PLANE + src, PLANE)],
                        planes_v.at[pl.ds((oc * B + bb) * PLANE, PLANE)],
                        sem_stage)
                    handles.append(h)
            for h in handles:
                h.wait()

        rb0 = jnp.maximum(item0 - g * NB, 0)
        rb1 = jnp.minimum(item0 + ITEMS_PER_W - g * NB, NB)
        ph = g // POOLED
        pw = g % POOLED
        ph_f = ph.astype(jnp.float32)
        pw_f = pw.astype(jnp.float32)

        @plsc.parallel_loop(rb0, rb1, 1, unroll=1)
        def rb_body(rb):
            item = g * NB + rb
            slot = item % RING

            for k in range(RING):
                @pl.when((slot == k) & (item - item0 >= RING))
                def _(k=k):
                    for oc in range(OC):
                        pltpu.make_async_copy(
                            out_hbm.at[oc, ph, pw, pl.ds(rb * L, L)],
                            outs[k].at[oc], sem_out.at[k]).wait()

            off = rb * L
            b_f = params_v[0, pl.ds(off, L)]
            cx = params_v[1, pl.ds(off, L)]
            cy = params_v[2, pl.ds(off, L)]
            bin_w = params_v[3, pl.ds(off, L)]
            bin_h = params_v[4, pl.ds(off, L)]
            half_w = params_v[5, pl.ds(off, L)]
            half_h = params_v[6, pl.ds(off, L)]
            cos_t = params_v[7, pl.ds(off, L)]
            sin_t = params_v[8, pl.ds(off, L)]
            base = b_f.astype(jnp.int32) * PLANE
            acc = [jnp.zeros((L,), jnp.float32) for _ in range(OC)]
            for s in range(SAMPLE * SAMPLE):
                sy_c = (s // SAMPLE + 0.5) / SAMPLE
                sx_c = (s % SAMPLE + 0.5) / SAMPLE
                xx = (pw_f + sx_c) * bin_w - half_w
                yy = (ph_f + sy_c) * bin_h - half_h
                x = xx * cos_t - yy * sin_t + cx
                y = xx * sin_t + yy * cos_t + cy
                valid = (y > -1.0) & (y < float(H)) & (x > -1.0) & (x < float(W))
                yc = jnp.clip(y, 0.0, H - 1.0)
                xc = jnp.clip(x, 0.0, W - 1.0)
                y0 = yc.astype(jnp.int32)
                x0 = xc.astype(jnp.int32)
                y1 = jnp.minimum(y0 + 1, H - 1)
                ly = yc - y0.astype(jnp.float32)
                lx = xc - x0.astype(jnp.float32)
                hy = 1.0 - ly
                hx = 1.0 - lx
                scl = jnp.where(valid, 1.0 / (SAMPLE * SAMPLE), 0.0)
                hy = hy * scl
                ly = ly * scl
                w00 = hy * hx
                w01 = hy * lx
                w10 = ly * hx
                w11 = ly * lx
                i00 = base + y0 * W + x0
                i10 = base + y1 * W + x0
                wtop = plsc.pack(w00, w01, format=plsc.PackFormat.INTERLEAVED)
                wbot = plsc.pack(w10, w11, format=plsc.PackFormat.INTERLEAVED)
                mask_hi = jnp.full((L,), -65536, jnp.int32)
                for oc in range(OC):
                    ref = planes_v.at[pl.ds(oc * B * PLANE, B * PLANE)]
                    gt = plsc.bitcast(plsc.load_gather(ref, [i00]),
                                      jnp.bfloat16)
                    gb = plsc.bitcast(plsc.load_gather(ref, [i10]),
                                      jnp.bfloat16)
                    sm = plsc.bitcast(gt * wtop + gb * wbot, jnp.int32)
                    lo = plsc.bitcast(sm << 16, jnp.float32)
                    hi = plsc.bitcast(sm & mask_hi, jnp.float32)
                    acc[oc] = acc[oc] + (lo + hi)
            for k in range(RING):
                @pl.when(slot == k)
                def _(k=k):
                    for oc in range(OC):
                        outs[k][oc, :] = acc[oc]
                    for oc in range(OC):
                        pltpu.async_copy(
                            outs[k].at[oc],
                            out_hbm.at[oc, ph, pw, pl.ds(rb * L, L)],
                            sem_out.at[k])

        return 0

    lax.fori_loop(g0, g1 + 1, g_body, 0)

    for k in range(RING):
        for oc in range(OC):
            pltpu.make_async_copy(out_hbm.at[oc, 0, 0, pl.ds(k * L, L)],
                                  outs[k].at[oc], sem_out.at[k]).wait()


def kernel(features, rois):
    rois_t = rois.T

    feat_t = features.transpose(0, 2, 3, 1)
    yblk = 16
    packed = pl.pallas_call(
        _pack_body,
        grid=(B, H // yblk),
        in_specs=[pl.BlockSpec((1, yblk, W, C), lambda b, i: (b, i, 0, 0))],
        out_specs=pl.BlockSpec((1, C, yblk // 2, 128),
                               lambda b, i: (b, 0, i, 0)),
        out_shape=jax.ShapeDtypeStruct((B, C, H * W // 128, 128), jnp.int32),
    )(feat_t)
    feat_flat = packed.reshape(B * C * PLANE)

    params = pl.pallas_call(
        _params_body,
        out_shape=jax.ShapeDtypeStruct((16, NROIS), jnp.float32),
    )(rois_t)

    mesh = plsc.VectorSubcoreMesh(core_axis_name="c", subcore_axis_name="s",
                                  num_cores=NC, num_subcores=NS)
    sc = functools.partial(
        pl.kernel,
        mesh=mesh,
        out_type=jax.ShapeDtypeStruct((OC, POOLED, 8, NROIS), jnp.float32),
        scratch_types=[
            pltpu.VMEM((PLANES_WORDS,), jnp.int32),
            pltpu.VMEM((16, NROIS), jnp.float32),
            pltpu.VMEM((OC, L), jnp.float32),
            pltpu.VMEM((OC, L), jnp.float32),
            pltpu.VMEM((OC, L), jnp.float32),
            pltpu.VMEM((OC, L), jnp.float32),
            pltpu.SemaphoreType.DMA,
            pltpu.SemaphoreType.DMA((RING,)),
        ],
        compiler_params=pltpu.CompilerParams(needs_layout_passes=False),
    )(_sc_body)
    out = sc(feat_flat, params)

    return out[:, :, :POOLED, :].transpose(3, 0, 1, 2)


# speedup vs baseline: 1.3278x; 1.0793x over previous
import functools

import jax
import jax.numpy as jnp
from jax import lax
from jax.experimental import pallas as pl
from jax.experimental.pallas import tpu as pltpu
from jax.experimental.pallas import tpu_sc as plsc

POOLED = 7
GROUP = 7
SCALE = 0.0625
SAMPLE = 4
B, C, H, W = 2, 490, 64, 64
OC = C // (GROUP * GROUP)
NROIS = 1024
NG = POOLED * POOLED
PLANE = H * W
PLANES_WORDS = B * OC * PLANE
NC, NS, L = 2, 16, 16
NW = NC * NS
NB = NROIS // L
ITEMS = NG * NB
ITEMS_PER_W = ITEMS // NW
RING = 4


def _pack_body(x_ref, out_ref):
    x = x_ref[...][0]
    xs = jnp.concatenate([x[:, 1:, :], x[:, 63:64, :]], axis=1)
    lo = jax.lax.bitcast_convert_type(x.astype(jnp.bfloat16),
                                      jnp.uint16).astype(jnp.uint32)
    hi = jax.lax.bitcast_convert_type(xs.astype(jnp.bfloat16),
                                      jnp.uint16).astype(jnp.uint32)
    wordn = jax.lax.bitcast_convert_type(lo | (hi << 16), jnp.int32)
    word = jnp.transpose(wordn, (2, 0, 1))
    w4 = word.reshape(C, 8, 2, 64)
    out_ref[...] = jnp.concatenate(
        [w4[:, :, 0, :], w4[:, :, 1, :]], axis=-1)[None]


def _params_body(rois_ref, out_ref):
    r = rois_ref[...]
    b = r[0]
    cx = r[1] * SCALE
    cy = r[2] * SCALE
    rw = jnp.maximum(r[3] * SCALE, 0.1)
    rh = jnp.maximum(r[4] * SCALE, 0.1)
    theta = r[5]
    bin_w = rw / POOLED
    bin_h = rh / POOLED
    half_w = rw / 2.0
    half_h = rh / 2.0
    cos_t = jnp.cos(theta)
    sin_t = jnp.sin(theta)
    rows = jnp.stack([b, cx, cy, bin_w, bin_h, half_w, half_h, cos_t, sin_t])
    pad = jnp.zeros((16 - rows.shape[0], NROIS), jnp.float32)
    out_ref[...] = jnp.concatenate([rows, pad], axis=0)


def _sc_body(feat_hbm, params_hbm, out_hbm, planes_v, params_v,
             ov0, ov1, ov2, ov3, sem_stage, sem_out):
    outs = [ov0, ov1, ov2, ov3]
    wid = lax.axis_index("s") * NC + lax.axis_index("c")
    pltpu.sync_copy(params_hbm, params_v)
    item0 = wid * ITEMS_PER_W
    g0 = item0 // NB
    g1 = (item0 + ITEMS_PER_W - 1) // NB

    def g_body(g, _):
        with jax.named_scope("stage_planes"):
            handles = []
            for bb in range(B):
                for oc in range(OC):
                    src = (bb * C + oc * GROUP * GROUP) * PLANE
                    h = pltpu.async_copy(
                        feat_hbm.at[pl.ds(g * PLANE + src, PLANE)],
                        planes_v.at[pl.ds((oc * B + bb) * PLANE, PLANE)],
                        sem_stage)
                    handles.append(h)
            for h in handles:
                h.wait()

        rb0 = jnp.maximum(item0 - g * NB, 0)
        rb1 = jnp.minimum(item0 + ITEMS_PER_W - g * NB, NB)
        ph = g // POOLED
        pw = g % POOLED
        ph_f = ph.astype(jnp.float32)
        pw_f = pw.astype(jnp.float32)

        @plsc.parallel_loop(rb0, rb1, 1, unroll=1)
        def rb_body(rb):
            item = g * NB + rb
            slot = item % RING

            for k in range(RING):
                @pl.when((slot == k) & (item - item0 >= RING))
                def _(k=k):
                    for oc in range(OC):
                        pltpu.make_async_copy(
                            out_hbm.at[oc, ph, pw, pl.ds(rb * L, L)],
                            outs[k].at[oc], sem_out.at[k]).wait()

            off = rb * L
            b_f = params_v[0, pl.ds(off, L)]
            cx = params_v[1, pl.ds(off, L)]
            cy = params_v[2, pl.ds(off, L)]
            bin_w = params_v[3, pl.ds(off, L)]
            bin_h = params_v[4, pl.ds(off, L)]
            half_w = params_v[5, pl.ds(off, L)]
            half_h = params_v[6, pl.ds(off, L)]
            cos_t = params_v[7, pl.ds(off, L)]
            sin_t = params_v[8, pl.ds(off, L)]
            base = b_f.astype(jnp.int32) * PLANE
            acc = [jnp.zeros((L,), jnp.float32) for _ in range(OC)]
            for s in range(SAMPLE * SAMPLE):
                sy_c = (s // SAMPLE + 0.5) / SAMPLE
                sx_c = (s % SAMPLE + 0.5) / SAMPLE
                xx = (pw_f + sx_c) * bin_w - half_w
                yy = (ph_f + sy_c) * bin_h - half_h
                x = xx * cos_t - yy * sin_t + cx
                y = xx * sin_t + yy * cos_t + cy
                valid = (y > -1.0) & (y < float(H)) & (x > -1.0) & (x < float(W))
                yc = jnp.clip(y, 0.0, H - 1.0)
                xc = jnp.clip(x, 0.0, W - 1.0)
                y0 = yc.astype(jnp.int32)
                x0 = xc.astype(jnp.int32)
                y1 = jnp.minimum(y0 + 1, H - 1)
                ly = yc - y0.astype(jnp.float32)
                lx = xc - x0.astype(jnp.float32)
                hy = 1.0 - ly
                hx = 1.0 - lx
                scl = jnp.where(valid, 1.0 / (SAMPLE * SAMPLE), 0.0)
                hy = hy * scl
                ly = ly * scl
                w00 = hy * hx
                w01 = hy * lx
                w10 = ly * hx
                w11 = ly * lx
                i00 = base + y0 * W + x0
                i10 = base + y1 * W + x0
                wtop = plsc.pack(w00, w01, format=plsc.PackFormat.INTERLEAVED)
                wbot = plsc.pack(w10, w11, format=plsc.PackFormat.INTERLEAVED)
                mask_hi = jnp.full((L,), -65536, jnp.int32)
                for oc in range(OC):
                    ref = planes_v.at[pl.ds(oc * B * PLANE, B * PLANE)]
                    gt = plsc.bitcast(plsc.load_gather(ref, [i00]),
                                      jnp.bfloat16)
                    gb = plsc.bitcast(plsc.load_gather(ref, [i10]),
                                      jnp.bfloat16)
                    sm = plsc.bitcast(gt * wtop + gb * wbot, jnp.int32)
                    lo = plsc.bitcast(sm << 16, jnp.float32)
                    hi = plsc.bitcast(sm & mask_hi, jnp.float32)
                    acc[oc] = acc[oc] + (lo + hi)
            for k in range(RING):
                @pl.when(slot == k)
                def _(k=k):
                    for oc in range(OC):
                        outs[k][oc, :] = acc[oc]
                    for oc in range(OC):
                        pltpu.async_copy(
                            outs[k].at[oc],
                            out_hbm.at[oc, ph, pw, pl.ds(rb * L, L)],
                            sem_out.at[k])

        return 0

    lax.fori_loop(g0, g1 + 1, g_body, 0)

    for k in range(RING):
        for oc in range(OC):
            pltpu.make_async_copy(out_hbm.at[oc, 0, 0, pl.ds(k * L, L)],
                                  outs[k].at[oc], sem_out.at[k]).wait()


def kernel(features, rois):
    rois_t = rois.T

    feat_t = features.transpose(0, 2, 3, 1)
    yblk = 16
    packed = pl.pallas_call(
        _pack_body,
        grid=(B, H // yblk),
        in_specs=[pl.BlockSpec((1, yblk, W, C), lambda b, i: (b, i, 0, 0))],
        out_specs=pl.BlockSpec((1, C, yblk // 2, 128),
                               lambda b, i: (b, 0, i, 0)),
        out_shape=jax.ShapeDtypeStruct((B, C, H * W // 128, 128), jnp.int32),
    )(feat_t)
    feat_flat = packed.reshape(B * C * PLANE)

    params = pl.pallas_call(
        _params_body,
        out_shape=jax.ShapeDtypeStruct((16, NROIS), jnp.float32),
    )(rois_t)

    mesh = plsc.VectorSubcoreMesh(core_axis_name="c", subcore_axis_name="s",
                                  num_cores=NC, num_subcores=NS)
    sc = functools.partial(
        pl.kernel,
        mesh=mesh,
        out_type=jax.ShapeDtypeStruct((OC, POOLED, 8, NROIS), jnp.float32),
        scratch_types=[
            pltpu.VMEM((PLANES_WORDS,), jnp.int32),
            pltpu.VMEM((16, NROIS), jnp.float32),
            pltpu.VMEM((OC, L), jnp.float32),
            pltpu.VMEM((OC, L), jnp.float32),
            pltpu.VMEM((OC, L), jnp.float32),
            pltpu.VMEM((OC, L), jnp.float32),
            pltpu.SemaphoreType.DMA,
            pltpu.SemaphoreType.DMA((RING,)),
        ],
        compiler_params=pltpu.CompilerParams(needs_layout_passes=False),
    )(_sc_body)
    out = sc(feat_flat, params)

    return out[:, :, :POOLED, :].transpose(3, 0, 1, 2)


# speedup vs baseline: 1.3427x; 1.0113x over previous
import functools

import jax
import jax.numpy as jnp
from jax import lax
from jax.experimental import pallas as pl
from jax.experimental.pallas import tpu as pltpu
from jax.experimental.pallas import tpu_sc as plsc

POOLED = 7
GROUP = 7
SCALE = 0.0625
SAMPLE = 4
B, C, H, W = 2, 490, 64, 64
OC = C // (GROUP * GROUP)
NROIS = 1024
NG = POOLED * POOLED
PLANE = H * W
PLANES_WORDS = B * OC * PLANE
NC, NS, L = 2, 16, 16
NW = NC * NS
NB = NROIS // L
ITEMS = NG * NB
ITEMS_PER_W = ITEMS // NW
RING = 4


def _pack_body(x_ref, out_ref):
    x = x_ref[...][0]
    xs = jnp.concatenate([x[:, 1:, :], x[:, 63:64, :]], axis=1)
    lo = jax.lax.bitcast_convert_type(x.astype(jnp.bfloat16),
                                      jnp.uint16).astype(jnp.uint32)
    hi = jax.lax.bitcast_convert_type(xs.astype(jnp.bfloat16),
                                      jnp.uint16).astype(jnp.uint32)
    wordn = jax.lax.bitcast_convert_type(lo | (hi << 16), jnp.int32)
    word = jnp.transpose(wordn, (2, 0, 1))
    w4 = word.reshape(C, 8, 2, 64)
    out_ref[...] = jnp.concatenate(
        [w4[:, :, 0, :], w4[:, :, 1, :]], axis=-1)[None]


def _params_body(rois_ref, out_ref):
    r = rois_ref[...]
    b = r[0]
    cx = r[1] * SCALE
    cy = r[2] * SCALE
    rw = jnp.maximum(r[3] * SCALE, 0.1)
    rh = jnp.maximum(r[4] * SCALE, 0.1)
    theta = r[5]
    bin_w = rw / POOLED
    bin_h = rh / POOLED
    half_w = rw / 2.0
    half_h = rh / 2.0
    cos_t = jnp.cos(theta)
    sin_t = jnp.sin(theta)
    rows = jnp.stack([b, cx, cy, bin_w, bin_h, half_w, half_h, cos_t, sin_t])
    pad = jnp.zeros((16 - rows.shape[0], NROIS), jnp.float32)
    out_ref[...] = jnp.concatenate([rows, pad], axis=0)


def _sc_body(feat_hbm, params_hbm, out_hbm, planes_v, params_v,
             ov0, ov1, ov2, ov3, sem_stage, sem_out):
    outs = [ov0, ov1, ov2, ov3]
    wid = lax.axis_index("s") * NC + lax.axis_index("c")
    pltpu.sync_copy(params_hbm, params_v)
    item0 = wid * ITEMS_PER_W
    g0 = item0 // NB
    g1 = (item0 + ITEMS_PER_W - 1) // NB

    def g_body(g, _):
        with jax.named_scope("stage_planes"):
            handles = []
            for bb in range(B):
                for oc in range(OC):
                    src = (bb * C + oc * GROUP * GROUP) * PLANE
                    h = pltpu.async_copy(
                        feat_hbm.at[pl.ds(g * PLANE + src, PLANE)],
                        planes_v.at[pl.ds((oc * B + bb) * PLANE, PLANE)],
                        sem_stage)
                    handles.append(h)
            for h in handles:
                h.wait()

        rb0 = jnp.maximum(item0 - g * NB, 0)
        rb1 = jnp.minimum(item0 + ITEMS_PER_W - g * NB, NB)
        ph = g // POOLED
        pw = g % POOLED
        ph_f = ph.astype(jnp.float32)
        pw_f = pw.astype(jnp.float32)

        @plsc.parallel_loop(rb0, rb1, 1, unroll=1)
        def rb_body(rb):
            item = g * NB + rb
            slot = item % RING

            for k in range(RING):
                @pl.when((slot == k) & (item - item0 >= RING))
                def _(k=k):
                    for oc in range(OC):
                        pltpu.make_async_copy(
                            out_hbm.at[oc, ph, pw, pl.ds(rb * L, L)],
                            outs[k].at[oc], sem_out.at[k]).wait()

            off = rb * L
            b_f = params_v[0, pl.ds(off, L)]
            cx = params_v[1, pl.ds(off, L)]
            cy = params_v[2, pl.ds(off, L)]
            bin_w = params_v[3, pl.ds(off, L)]
            bin_h = params_v[4, pl.ds(off, L)]
            half_w = params_v[5, pl.ds(off, L)]
            half_h = params_v[6, pl.ds(off, L)]
            cos_t = params_v[7, pl.ds(off, L)]
            sin_t = params_v[8, pl.ds(off, L)]
            base = b_f.astype(jnp.int32) * PLANE
            acc = [jnp.zeros((L,), jnp.float32) for _ in range(OC)]
            xca, xsa, ysb, ycb = [], [], [], []
            for a in range(SAMPLE):
                xx = (pw_f + (a + 0.5) / SAMPLE) * bin_w - half_w
                xca.append(xx * cos_t + cx)
                xsa.append(xx * sin_t + cy)
            for bq in range(SAMPLE):
                yy = (ph_f + (bq + 0.5) / SAMPLE) * bin_h - half_h
                ysb.append(yy * sin_t)
                ycb.append(yy * cos_t)
            for s in range(SAMPLE * SAMPLE):
                x = xca[s % SAMPLE] - ysb[s // SAMPLE]
                y = xsa[s % SAMPLE] + ycb[s // SAMPLE]
                valid = (y > -1.0) & (y < float(H)) & (x > -1.0) & (x < float(W))
                yc = jnp.clip(y, 0.0, H - 1.0)
                xc = jnp.clip(x, 0.0, W - 1.0)
                y0 = yc.astype(jnp.int32)
                x0 = xc.astype(jnp.int32)
                y1 = jnp.minimum(y0 + 1, H - 1)
                ly = yc - y0.astype(jnp.float32)
                lx = xc - x0.astype(jnp.float32)
                hy = 1.0 - ly
                hx = 1.0 - lx
                scl = jnp.where(valid, 1.0 / (SAMPLE * SAMPLE), 0.0)
                hy = hy * scl
                ly = ly * scl
                w00 = hy * hx
                w01 = hy * lx
                w10 = ly * hx
                w11 = ly * lx
                i00 = base + y0 * W + x0
                i10 = base + y1 * W + x0
                wtop = plsc.pack(w00, w01, format=plsc.PackFormat.INTERLEAVED)
                wbot = plsc.pack(w10, w11, format=plsc.PackFormat.INTERLEAVED)
                mask_hi = jnp.full((L,), -65536, jnp.int32)
                for oc in range(OC):
                    ref = planes_v.at[pl.ds(oc * B * PLANE, B * PLANE)]
                    gt = plsc.bitcast(plsc.load_gather(ref, [i00]),
                                      jnp.bfloat16)
                    gb = plsc.bitcast(plsc.load_gather(ref, [i10]),
                                      jnp.bfloat16)
                    sm = plsc.bitcast(gt * wtop + gb * wbot, jnp.int32)
                    lo = plsc.bitcast(sm << 16, jnp.float32)
                    hi = plsc.bitcast(sm & mask_hi, jnp.float32)
                    acc[oc] = acc[oc] + (lo + hi)
            for k in range(RING):
                @pl.when(slot == k)
                def _(k=k):
                    for oc in range(OC):
                        outs[k][oc, :] = acc[oc]
                    for oc in range(OC):
                        pltpu.async_copy(
                            outs[k].at[oc],
                            out_hbm.at[oc, ph, pw, pl.ds(rb * L, L)],
                            sem_out.at[k])

        return 0

    lax.fori_loop(g0, g1 + 1, g_body, 0)

    for k in range(RING):
        for oc in range(OC):
            pltpu.make_async_copy(out_hbm.at[oc, 0, 0, pl.ds(k * L, L)],
                                  outs[k].at[oc], sem_out.at[k]).wait()


def kernel(features, rois):
    rois_t = rois.T

    feat_t = features.transpose(0, 2, 3, 1)
    yblk = 16
    packed = pl.pallas_call(
        _pack_body,
        grid=(B, H // yblk),
        in_specs=[pl.BlockSpec((1, yblk, W, C), lambda b, i: (b, i, 0, 0))],
        out_specs=pl.BlockSpec((1, C, yblk // 2, 128),
                               lambda b, i: (b, 0, i, 0)),
        out_shape=jax.ShapeDtypeStruct((B, C, H * W // 128, 128), jnp.int32),
    )(feat_t)
    feat_flat = packed.reshape(B * C * PLANE)

    params = pl.pallas_call(
        _params_body,
        out_shape=jax.ShapeDtypeStruct((16, NROIS), jnp.float32),
    )(rois_t)

    mesh = plsc.VectorSubcoreMesh(core_axis_name="c", subcore_axis_name="s",
                                  num_cores=NC, num_subcores=NS)
    sc = functools.partial(
        pl.kernel,
        mesh=mesh,
        out_type=jax.ShapeDtypeStruct((OC, POOLED, 8, NROIS), jnp.float32),
        scratch_types=[
            pltpu.VMEM((PLANES_WORDS,), jnp.int32),
            pltpu.VMEM((16, NROIS), jnp.float32),
            pltpu.VMEM((OC, L), jnp.float32),
            pltpu.VMEM((OC, L), jnp.float32),
            pltpu.VMEM((OC, L), jnp.float32),
            pltpu.VMEM((OC, L), jnp.float32),
            pltpu.SemaphoreType.DMA,
            pltpu.SemaphoreType.DMA((RING,)),
        ],
        compiler_params=pltpu.CompilerParams(needs_layout_passes=False),
    )(_sc_body)
    out = sc(feat_flat, params)

    return out[:, :, :POOLED, :].transpose(3, 0, 1, 2)


# speedup vs baseline: 1.3494x; 1.0050x over previous
import functools

import jax
import jax.numpy as jnp
from jax import lax
from jax.experimental import pallas as pl
from jax.experimental.pallas import tpu as pltpu
from jax.experimental.pallas import tpu_sc as plsc

POOLED = 7
GROUP = 7
SCALE = 0.0625
SAMPLE = 4
B, C, H, W = 2, 490, 64, 64
OC = C // (GROUP * GROUP)
NROIS = 1024
NG = POOLED * POOLED
PLANE = H * W
PLANES_WORDS = B * OC * PLANE
NC, NS, L = 2, 16, 16
NW = NC * NS
NB = NROIS // L
ITEMS = NG * NB
ITEMS_PER_W = ITEMS // NW
RING = 4


def _pack_body(x_ref, out_ref):
    x = x_ref[...][0]
    xb = jax.lax.bitcast_convert_type(x.astype(jnp.bfloat16), jnp.uint16)
    xsb = jnp.concatenate([xb[:, 1:, :], xb[:, 63:64, :]], axis=1)
    wordn = jax.lax.bitcast_convert_type(
        xb.astype(jnp.uint32) | (xsb.astype(jnp.uint32) << 16), jnp.int32)
    word = jnp.transpose(wordn, (2, 0, 1))
    w4 = word.reshape(C, 8, 2, 64)
    out_ref[...] = jnp.concatenate(
        [w4[:, :, 0, :], w4[:, :, 1, :]], axis=-1)[None]


def _params_body(rois_ref, out_ref):
    r = rois_ref[...]
    b = r[0]
    cx = r[1] * SCALE
    cy = r[2] * SCALE
    rw = jnp.maximum(r[3] * SCALE, 0.1)
    rh = jnp.maximum(r[4] * SCALE, 0.1)
    theta = r[5]
    bin_w = rw / POOLED
    bin_h = rh / POOLED
    half_w = rw / 2.0
    half_h = rh / 2.0
    cos_t = jnp.cos(theta)
    sin_t = jnp.sin(theta)
    rows = jnp.stack([b, cx, cy, bin_w, bin_h, half_w, half_h, cos_t, sin_t])
    pad = jnp.zeros((16 - rows.shape[0], NROIS), jnp.float32)
    out_ref[...] = jnp.concatenate([rows, pad], axis=0)


def _sc_body(feat_hbm, params_hbm, out_hbm, planes_v, params_v,
             ov0, ov1, ov2, ov3, sem_stage, sem_out):
    outs = [ov0, ov1, ov2, ov3]
    wid = lax.axis_index("s") * NC + lax.axis_index("c")
    pltpu.sync_copy(params_hbm, params_v)
    item0 = wid * ITEMS_PER_W
    g0 = item0 // NB
    g1 = (item0 + ITEMS_PER_W - 1) // NB

    def g_body(g, _):
        with jax.named_scope("stage_planes"):
            handles = []
            for bb in range(B):
                for oc in range(OC):
                    src = (bb * C + oc * GROUP * GROUP) * PLANE
                    h = pltpu.async_copy(
                        feat_hbm.at[pl.ds(g * PLANE + src, PLANE)],
                        planes_v.at[pl.ds((oc * B + bb) * PLANE, PLANE)],
                        sem_stage)
                    handles.append(h)
            for h in handles:
                h.wait()

        rb0 = jnp.maximum(item0 - g * NB, 0)
        rb1 = jnp.minimum(item0 + ITEMS_PER_W - g * NB, NB)
        ph = g // POOLED
        pw = g % POOLED
        ph_f = ph.astype(jnp.float32)
        pw_f = pw.astype(jnp.float32)

        @plsc.parallel_loop(rb0, rb1, 1, unroll=1)
        def rb_body(rb):
            item = g * NB + rb
            slot = item % RING

            for k in range(RING):
                @pl.when((slot == k) & (item - item0 >= RING))
                def _(k=k):
                    for oc in range(OC):
                        pltpu.make_async_copy(
                            out_hbm.at[oc, ph, pw, pl.ds(rb * L, L)],
                            outs[k].at[oc], sem_out.at[k]).wait()

            off = rb * L
            b_f = params_v[0, pl.ds(off, L)]
            cx = params_v[1, pl.ds(off, L)]
            cy = params_v[2, pl.ds(off, L)]
            bin_w = params_v[3, pl.ds(off, L)]
            bin_h = params_v[4, pl.ds(off, L)]
            half_w = params_v[5, pl.ds(off, L)]
            half_h = params_v[6, pl.ds(off, L)]
            cos_t = params_v[7, pl.ds(off, L)]
            sin_t = params_v[8, pl.ds(off, L)]
            base = b_f.astype(jnp.int32) * PLANE
            acc = [jnp.zeros((L,), jnp.float32) for _ in range(OC)]
            xca, xsa, ysb, ycb = [], [], [], []
            for a in range(SAMPLE):
                xx = (pw_f + (a + 0.5) / SAMPLE) * bin_w - half_w
                xca.append(xx * cos_t + cx)
                xsa.append(xx * sin_t + cy)
            for bq in range(SAMPLE):
                yy = (ph_f + (bq + 0.5) / SAMPLE) * bin_h - half_h
                ysb.append(yy * sin_t)
                ycb.append(yy * cos_t)
            for s in range(SAMPLE * SAMPLE):
                x = xca[s % SAMPLE] - ysb[s // SAMPLE]
                y = xsa[s % SAMPLE] + ycb[s // SAMPLE]
                valid = (y > -1.0) & (y < float(H)) & (x > -1.0) & (x < float(W))
                yc = jnp.clip(y, 0.0, H - 1.0)
                xc = jnp.clip(x, 0.0, W - 1.0)
                y0 = yc.astype(jnp.int32)
                x0 = xc.astype(jnp.int32)
                y1 = jnp.minimum(y0 + 1, H - 1)
                ly = yc - y0.astype(jnp.float32)
                lx = xc - x0.astype(jnp.float32)
                hy = 1.0 - ly
                hx = 1.0 - lx
                scl = jnp.where(valid, 1.0 / (SAMPLE * SAMPLE), 0.0)
                hy = hy * scl
                ly = ly * scl
                w00 = hy * hx
                w01 = hy * lx
                w10 = ly * hx
                w11 = ly * lx
                i00 = base + y0 * W + x0
                i10 = base + y1 * W + x0
                wtop = plsc.pack(w00, w01, format=plsc.PackFormat.INTERLEAVED)
                wbot = plsc.pack(w10, w11, format=plsc.PackFormat.INTERLEAVED)
                mask_hi = jnp.full((L,), -65536, jnp.int32)
                for oc in range(OC):
                    ref = planes_v.at[pl.ds(oc * B * PLANE, B * PLANE)]
                    gt = plsc.bitcast(plsc.load_gather(ref, [i00]),
                                      jnp.bfloat16)
                    gb = plsc.bitcast(plsc.load_gather(ref, [i10]),
                                      jnp.bfloat16)
                    sm = plsc.bitcast(gt * wtop + gb * wbot, jnp.int32)
                    lo = plsc.bitcast(sm << 16, jnp.float32)
                    hi = plsc.bitcast(sm & mask_hi, jnp.float32)
                    acc[oc] = acc[oc] + (lo + hi)
            for k in range(RING):
                @pl.when(slot == k)
                def _(k=k):
                    for oc in range(OC):
                        outs[k][oc, :] = acc[oc]
                    for oc in range(OC):
                        pltpu.async_copy(
                            outs[k].at[oc],
                            out_hbm.at[oc, ph, pw, pl.ds(rb * L, L)],
                            sem_out.at[k])

        return 0

    lax.fori_loop(g0, g1 + 1, g_body, 0)

    for k in range(RING):
        for oc in range(OC):
            pltpu.make_async_copy(out_hbm.at[oc, 0, 0, pl.ds(k * L, L)],
                                  outs[k].at[oc], sem_out.at[k]).wait()


def kernel(features, rois):
    rois_t = rois.T

    feat_t = features.transpose(0, 2, 3, 1)
    yblk = 16
    packed = pl.pallas_call(
        _pack_body,
        grid=(B, H // yblk),
        in_specs=[pl.BlockSpec((1, yblk, W, C), lambda b, i: (b, i, 0, 0))],
        out_specs=pl.BlockSpec((1, C, yblk // 2, 128),
                               lambda b, i: (b, 0, i, 0)),
        out_shape=jax.ShapeDtypeStruct((B, C, H * W // 128, 128), jnp.int32),
    )(feat_t)
    feat_flat = packed.reshape(B * C * PLANE)

    params = pl.pallas_call(
        _params_body,
        out_shape=jax.ShapeDtypeStruct((16, NROIS), jnp.float32),
    )(rois_t)

    mesh = plsc.VectorSubcoreMesh(core_axis_name="c", subcore_axis_name="s",
                                  num_cores=NC, num_subcores=NS)
    sc = functools.partial(
        pl.kernel,
        mesh=mesh,
        out_type=jax.ShapeDtypeStruct((OC, POOLED, 8, NROIS), jnp.float32),
        scratch_types=[
            pltpu.VMEM((PLANES_WORDS,), jnp.int32),
            pltpu.VMEM((16, NROIS), jnp.float32),
            pltpu.VMEM((OC, L), jnp.float32),
            pltpu.VMEM((OC, L), jnp.float32),
            pltpu.VMEM((OC, L), jnp.float32),
            pltpu.VMEM((OC, L), jnp.float32),
            pltpu.SemaphoreType.DMA,
            pltpu.SemaphoreType.DMA((RING,)),
        ],
        compiler_params=pltpu.CompilerParams(needs_layout_passes=False),
    )(_sc_body)
    out = sc(feat_flat, params)

    return out[:, :, :POOLED, :].transpose(3, 0, 1, 2)
